# baseline (speedup 1.0000x reference)
import functools

import jax
import jax.numpy as jnp
from jax import lax
from jax.experimental import pallas as pl
from jax.experimental.pallas import tpu as pltpu
from jax.experimental.pallas import tpu_sc as plsc

INPUT_DIM = 2048
HIDDEN_DIM = 32768
K = 64
BATCH = 32

HB = 2048
N_BLK = HIDDEN_DIM // HB

HALF = HIDDEN_DIM // 2
NH_BLK = HALF // HB

L = 16
NVREG_H = HALF // L
CMAX_H = 256
CAP_H = HALF + L


def _enc_body(x_ref, w_ref, b_ref, f_ref):
    acc = jax.lax.dot_general(
        x_ref[...], w_ref[...],
        (((1,), (1,)), ((), ())),
        preferred_element_type=jnp.float32,
    )
    f_ref[...] = jnp.maximum(acc + b_ref[...], 0.0) + 0.0


def _popcnt(mask):
    return plsc.all_reduce_population_count(mask)[0]


def _make_topk_half(idx_off):
    def _topk_half(f_hbm, v_out, i_out, row_v, cmax_v, cand_v, cidx_v,
                   ov_v, oi_v):
        wid = lax.axis_index("s") * 2 + lax.axis_index("c")
        pltpu.sync_copy(f_hbm.at[wid], row_v)

        iota = lax.broadcasted_iota(jnp.int32, (L,), 0)

        def cmax_group(g, _):
            def inner(j, acc):
                return jnp.maximum(acc, row_v[pl.ds((g * 64 + j) * L, L)])
            m = lax.fori_loop(0, 64, inner, jnp.zeros((L,), jnp.int32))
            cmax_v[pl.ds(g * L, L)] = m
            return 0

        lax.fori_loop(0, 16, cmax_group, 0)

        def lb_step(i, t):
            cand = t | (jnp.int32(1) << (30 - i))
            candv = jnp.zeros((L,), jnp.int32) + cand

            def cnt_step(v, c):
                return c + _popcnt(cmax_v[pl.ds(v * L, L)] >= candv)

            cnt = lax.fori_loop(0, CMAX_H // L, cnt_step, jnp.int32(0))
            return jnp.where(cnt >= K, cand, t)

        lb = lax.fori_loop(0, 16, lb_step, jnp.int32(0))
        lbv = jnp.zeros((L,), jnp.int32) + lb

        def flt_group(g, wp):
            base = g * 8 * L
            vs = [row_v[pl.ds(base + j * L, L)] for j in range(8)]
            hit = vs[0] >= lbv
            for j in range(1, 8):
                hit = hit | (vs[j] >= lbv)
            nhit = _popcnt(hit)

            def compact(wp):
                for j in range(8):
                    m = vs[j] >= lbv
                    plsc.store_compressed(
                        cand_v.at[pl.ds(wp, L)], vs[j], mask=m)
                    plsc.store_compressed(
                        cidx_v.at[pl.ds(wp, L)],
                        idx_off + base + j * L + iota, mask=m)
                    wp = wp + _popcnt(m)
                return wp

            return lax.cond(nhit > 0, compact, lambda w: w, wp)

        wp = lax.fori_loop(0, NVREG_H // 8, flt_group, jnp.int32(0))
        nv = (wp + L - 1) // L
        wpv = jnp.zeros((L,), jnp.int32) + wp
        loc = lax.broadcasted_iota(jnp.int32, (L,), 0)

        def val_step(i, t):
            cand = t | (jnp.int32(1) << (30 - i))
            candv = jnp.zeros((L,), jnp.int32) + cand

            def cnt_step(v, c):
                valid = (v * L + loc) < wpv
                ge = (cand_v[pl.ds(v * L, L)] >= candv) & valid
                return c + _popcnt(ge)

            cnt = lax.fori_loop(0, nv, cnt_step, jnp.int32(0))
            return jnp.where(cnt >= K, cand, t)

        t = lax.fori_loop(0, 31, val_step, jnp.int32(0))
        tv = jnp.zeros((L,), jnp.int32) + t

        def gt_step(v, c):
            valid = (v * L + loc) < wpv
            gt = (cand_v[pl.ds(v * L, L)] > tv) & valid
            return c + _popcnt(gt)

        cnt_gt = lax.fori_loop(0, nv, gt_step, jnp.int32(0))
        m = K - cnt_gt

        one_v = jnp.full((L,), 1, jnp.int32)
        zero_v = jnp.zeros((L,), jnp.int32)
        neg1_v = jnp.full((L,), -1, jnp.int32)

        def tie_step(v, carry):
            cbefore, cfound = carry
            valid = (v * L + loc) < wpv
            eq = (cand_v[pl.ds(v * L, L)] == tv) & valid
            cs = plsc.cumsum(jnp.where(eq, one_v, zero_v))
            kv = jnp.zeros((L,), jnp.int32) + (m - cbefore)
            hitlane = eq & (cs == kv)
            idxv = cidx_v[pl.ds(v * L, L)]
            cnd = plsc.cummax(jnp.where(hitlane, idxv, neg1_v))[L - 1]
            return cbefore + _popcnt(eq), jnp.maximum(cfound, cnd)

        _, cco = lax.fori_loop(0, nv, tie_step, (jnp.int32(0), jnp.int32(-1)))
        ccv = jnp.zeros((L,), jnp.int32) + cco

        def ext_step(v, wo):
            valid = (v * L + loc) < wpv
            cv = cand_v[pl.ds(v * L, L)]
            iv = cidx_v[pl.ds(v * L, L)]
            keep = ((cv > tv) | ((cv == tv) & (iv <= ccv))) & valid
            plsc.store_compressed(ov_v.at[pl.ds(wo, L)], cv, mask=keep)
            plsc.store_compressed(oi_v.at[pl.ds(wo, L)], iv, mask=keep)
            return wo + _popcnt(keep)

        lax.fori_loop(0, nv, ext_step, jnp.int32(0))

        pltpu.sync_copy(ov_v.at[pl.ds(0, K)], v_out.at[pl.ds(wid * K, K)])
        pltpu.sync_copy(oi_v.at[pl.ds(0, K)], i_out.at[pl.ds(wid * K, K)])

    return _topk_half


def _dec_body(fa_ref, fb_ref, va_ref, ia_ref, vb_ref, ib_ref, w_ref, o_ref,
              t_s, c_s):
    i = pl.program_id(0)

    @pl.when(i == 0)
    def _():
        cv = jnp.concatenate([va_ref[...], vb_ref[...]], axis=1)
        iv = jnp.concatenate([ia_ref[...], ib_ref[...]], axis=1)

        def val_step(s, t):
            cand = t | (jnp.int32(1) << (30 - s))
            cnt = jnp.sum((cv >= cand).astype(jnp.int32), axis=1,
                          keepdims=True)
            return jnp.where(cnt >= K, cand, t)

        t = lax.fori_loop(0, 31, val_step, jnp.zeros((BATCH, 1), jnp.int32))
        cnt_gt = jnp.sum((cv > t).astype(jnp.int32), axis=1, keepdims=True)
        m = K - cnt_gt
        eq = (cv == t)

        def idx_step(s, c):
            cand = c | (jnp.int32(1) << (15 - s))
            cnt = jnp.sum((eq & (iv < cand)).astype(jnp.int32), axis=1,
                          keepdims=True)
            return jnp.where(cnt < m, cand, c)

        j = lax.fori_loop(0, 16, idx_step, jnp.zeros((BATCH, 1), jnp.int32))
        t_s[...] = t
        c_s[...] = j

    f = jnp.where(i < NH_BLK, fa_ref[...], fb_ref[...])
    fbits = jax.lax.bitcast_convert_type(f, jnp.int32)
    t = t_s[...]
    c = c_s[...]
    idx = i * HB + jax.lax.broadcasted_iota(jnp.int32, fbits.shape, 1)
    keep = (fbits > t) | ((fbits == t) & (idx <= c))
    vals = jnp.where(keep, f, 0.0)
    part = jax.lax.dot_general(
        vals, w_ref[...],
        (((1,), (1,)), ((), ())),
        preferred_element_type=jnp.float32,
    )

    @pl.when(i == 0)
    def _():
        o_ref[...] = jnp.zeros_like(o_ref)

    o_ref[...] += part


def _enc_call(x, W_enc, b2d, half):
    off = half * NH_BLK
    return pl.pallas_call(
        _enc_body,
        grid=(NH_BLK,),
        in_specs=[
            pl.BlockSpec((BATCH, INPUT_DIM), lambda i: (0, 0)),
            pl.BlockSpec((HB, INPUT_DIM), lambda i: (i + off, 0)),
            pl.BlockSpec((1, HB), lambda i: (0, i + off)),
        ],
        out_specs=pl.BlockSpec((BATCH, HB), lambda i: (0, i)),
        out_shape=jax.ShapeDtypeStruct((BATCH, HALF), jnp.float32),
    )(x, W_enc, b2d)


def _sc_half_call(fbits, idx_off):
    mesh = plsc.VectorSubcoreMesh(core_axis_name="c", subcore_axis_name="s")
    return pl.kernel(
        _make_topk_half(idx_off),
        mesh=mesh,
        out_type=[
            jax.ShapeDtypeStruct((BATCH * K,), jnp.int32),
            jax.ShapeDtypeStruct((BATCH * K,), jnp.int32),
        ],
        scratch_types=[
            pltpu.VMEM((HALF,), jnp.int32),
            pltpu.VMEM((CMAX_H,), jnp.int32),
            pltpu.VMEM((CAP_H,), jnp.int32),
            pltpu.VMEM((CAP_H,), jnp.int32),
            pltpu.VMEM((K + L,), jnp.int32),
            pltpu.VMEM((K + L,), jnp.int32),
        ],
        compiler_params=pltpu.CompilerParams(
            needs_layout_passes=False, skip_device_barrier=True),
    )(fbits)


@jax.jit
def kernel(x, W_enc, b_enc, W_dec):
    b2d = b_enc.reshape(1, HIDDEN_DIM)

    feats_a = _enc_call(x, W_enc, b2d, 0)
    feats_b = _enc_call(x, W_enc, b2d, 1)

    va, ia = _sc_half_call(jax.lax.bitcast_convert_type(feats_a, jnp.int32), 0)
    vb, ib = _sc_half_call(
        jax.lax.bitcast_convert_type(feats_b, jnp.int32), HALF)
    va, ia, vb, ib = (z.reshape(BATCH, K) for z in (va, ia, vb, ib))

    recon = pl.pallas_call(
        _dec_body,
        grid=(N_BLK,),
        in_specs=[
            pl.BlockSpec((BATCH, HB),
                         lambda i: (0, jnp.minimum(i, NH_BLK - 1))),
            pl.BlockSpec((BATCH, HB),
                         lambda i: (0, jnp.maximum(i - NH_BLK, 0))),
            pl.BlockSpec((BATCH, K), lambda i: (0, 0)),
            pl.BlockSpec((BATCH, K), lambda i: (0, 0)),
            pl.BlockSpec((BATCH, K), lambda i: (0, 0)),
            pl.BlockSpec((BATCH, K), lambda i: (0, 0)),
            pl.BlockSpec((INPUT_DIM, HB), lambda i: (0, i)),
        ],
        out_specs=pl.BlockSpec((BATCH, INPUT_DIM), lambda i: (0, 0)),
        out_shape=jax.ShapeDtypeStruct((BATCH, INPUT_DIM), jnp.float32),
        scratch_shapes=[
            pltpu.VMEM((BATCH, 1), jnp.int32),
            pltpu.VMEM((BATCH, 1), jnp.int32),
        ],
    )(feats_a, feats_b, va, ia, vb, ib, W_dec)

    return recon


# speedup vs baseline: 1.0486x; 1.0486x over previous
import functools

import jax
import jax.numpy as jnp
from jax.experimental import pallas as pl

INPUT_DIM = 2048
HIDDEN_DIM = 32768
K = 64
BATCH = 32

HB = 2048
N_BLK = HIDDEN_DIM // HB


def _enc_body(x_ref, w_ref, b_ref, f_ref):
    acc = jax.lax.dot_general(
        x_ref[...], w_ref[...],
        (((1,), (1,)), ((), ())),
        preferred_element_type=jnp.float32,
    )
    f_ref[...] = jnp.maximum(acc + b_ref[...], 0.0) + 0.0


def _topk_body(f_ref, t_ref, c_ref):
    fb = jax.lax.bitcast_convert_type(f_ref[...], jnp.int32)
    hi = (fb >> 16).astype(jnp.int16)

    def hi_step(i, h32):
        cand32 = h32 | (jnp.int32(1) << (14 - i))
        cand16 = cand32.astype(jnp.int16)
        cnt = jnp.sum((hi >= cand16).astype(jnp.int32), axis=1, keepdims=True)
        return jnp.where(cnt >= K, cand32, h32)

    h32 = jax.lax.fori_loop(0, 15, hi_step, jnp.zeros((BATCH, 1), jnp.int32))
    h16 = h32.astype(jnp.int16)

    eq_hi = (hi == h16)
    lo_s = ((fb & 0xFFFF) - 32768).astype(jnp.int16)
    sel = jnp.where(eq_hi, lo_s, jnp.full_like(lo_s, -32768))
    cnt_hi_gt = jnp.sum((hi > h16).astype(jnp.int32), axis=1, keepdims=True)

    def lo_step(i, u):
        cand = u | (jnp.int32(1) << (15 - i))
        candv = (cand - 32768).astype(jnp.int16)
        cnt = cnt_hi_gt + jnp.sum((sel >= candv).astype(jnp.int32), axis=1,
                                  keepdims=True)
        return jnp.where(cnt >= K, cand, u)

    u = jax.lax.fori_loop(0, 16, lo_step, jnp.zeros((BATCH, 1), jnp.int32))
    t = (h32 << 16) | u

    cnt_gt = jnp.sum((fb > t).astype(jnp.int32), axis=1, keepdims=True)
    m = K - cnt_gt

    eq = (fb == t)
    cnt_eq = jnp.sum(eq.astype(jnp.int32), axis=1, keepdims=True)
    idx = jax.lax.broadcasted_iota(jnp.int32, fb.shape, 1)

    c_fast = jnp.max(jnp.where(eq, idx, -1), axis=1, keepdims=True)
    t_ref[...] = t
    c_ref[...] = c_fast

    @pl.when(jnp.max(cnt_eq - m) > 0)
    def _():
        def idx_step(i, c):
            shift = 15 - i
            cand = c | (jnp.int32(1) << shift)
            cnt = jnp.sum((eq & (idx < cand)).astype(jnp.int32), axis=1,
                          keepdims=True)
            return jnp.where(cnt < m, cand, c)

        j = jax.lax.fori_loop(0, 16, idx_step,
                              jnp.zeros((BATCH, 1), jnp.int32))
        c_ref[...] = j


def _dec_body(f_ref, t_ref, c_ref, w_ref, o_ref):
    i = pl.program_id(0)
    fb = jax.lax.bitcast_convert_type(f_ref[...], jnp.int32)
    t = t_ref[...]
    c = c_ref[...]
    idx = i * HB + jax.lax.broadcasted_iota(jnp.int32, fb.shape, 1)
    keep = (fb > t) | ((fb == t) & (idx <= c))
    vals = jnp.where(keep, f_ref[...], 0.0)
    part = jax.lax.dot_general(
        vals, w_ref[...],
        (((1,), (1,)), ((), ())),
        preferred_element_type=jnp.float32,
    )

    @pl.when(i == 0)
    def _():
        o_ref[...] = jnp.zeros_like(o_ref)

    o_ref[...] += part


@jax.jit
def kernel(x, W_enc, b_enc, W_dec):
    b2d = b_enc.reshape(1, HIDDEN_DIM)

    feats = pl.pallas_call(
        _enc_body,
        grid=(N_BLK,),
        in_specs=[
            pl.BlockSpec((BATCH, INPUT_DIM), lambda i: (0, 0)),
            pl.BlockSpec((HB, INPUT_DIM), lambda i: (i, 0)),
            pl.BlockSpec((1, HB), lambda i: (0, i)),
        ],
        out_specs=pl.BlockSpec((BATCH, HB), lambda i: (0, i)),
        out_shape=jax.ShapeDtypeStruct((BATCH, HIDDEN_DIM), jnp.float32),
    )(x, W_enc, b2d)

    tbits, cut = pl.pallas_call(
        _topk_body,
        in_specs=[pl.BlockSpec((BATCH, HIDDEN_DIM), lambda: (0, 0))],
        out_specs=[
            pl.BlockSpec((BATCH, 1), lambda: (0, 0)),
            pl.BlockSpec((BATCH, 1), lambda: (0, 0)),
        ],
        out_shape=[
            jax.ShapeDtypeStruct((BATCH, 1), jnp.int32),
            jax.ShapeDtypeStruct((BATCH, 1), jnp.int32),
        ],
    )(feats)

    recon = pl.pallas_call(
        _dec_body,
        grid=(N_BLK,),
        in_specs=[
            pl.BlockSpec((BATCH, HB), lambda i: (0, i)),
            pl.BlockSpec((BATCH, 1), lambda i: (0, 0)),
            pl.BlockSpec((BATCH, 1), lambda i: (0, 0)),
            pl.BlockSpec((INPUT_DIM, HB), lambda i: (0, i)),
        ],
        out_specs=pl.BlockSpec((BATCH, INPUT_DIM), lambda i: (0, 0)),
        out_shape=jax.ShapeDtypeStruct((BATCH, INPUT_DIM), jnp.float32),
    )(feats, tbits, cut, W_dec)

    return recon


# speedup vs baseline: 1.1565x; 1.1029x over previous
import functools

import jax
import jax.numpy as jnp
from jax.experimental import pallas as pl
from jax.experimental.pallas import tpu as pltpu

INPUT_DIM = 2048
HIDDEN_DIM = 32768
K = 64
BATCH = 32

HB = 2048
N_BLK = HIDDEN_DIM // HB


def _enc_body(x_ref, w_ref, b_ref, f_ref):
    acc = jax.lax.dot_general(
        x_ref[...], w_ref[...],
        (((1,), (1,)), ((), ())),
        preferred_element_type=jnp.float32,
    )
    f_ref[...] = jnp.maximum(acc + b_ref[...], 0.0) + 0.0


def _dec_body(f_ref, w_ref, o_ref, t_s, c_s):
    i = pl.program_id(0)

    @pl.when(i == 0)
    def _():
        fb = jax.lax.bitcast_convert_type(f_ref[...], jnp.int32)

        def val_step(s, t):
            cand = t | (jnp.int32(1) << (30 - s))
            cnt = jnp.sum((fb >= cand).astype(jnp.int32), axis=1,
                          keepdims=True)
            return jnp.where(cnt >= K, cand, t)

        t = jax.lax.fori_loop(0, 31, val_step,
                              jnp.zeros((BATCH, 1), jnp.int32))

        cnt_gt = jnp.sum((fb > t).astype(jnp.int32), axis=1, keepdims=True)
        m = K - cnt_gt

        eq = (fb == t)
        cnt_eq = jnp.sum(eq.astype(jnp.int32), axis=1, keepdims=True)
        idx = jax.lax.broadcasted_iota(jnp.int32, fb.shape, 1)

        t_s[...] = t
        c_s[...] = jnp.max(jnp.where(eq, idx, -1), axis=1, keepdims=True)

        @pl.when(jnp.max(cnt_eq - m) > 0)
        def _():
            def idx_step(s, c):
                cand = c | (jnp.int32(1) << (15 - s))
                cnt = jnp.sum((eq & (idx < cand)).astype(jnp.int32), axis=1,
                              keepdims=True)
                return jnp.where(cnt < m, cand, c)

            c_s[...] = jax.lax.fori_loop(0, 16, idx_step,
                                         jnp.zeros((BATCH, 1), jnp.int32))

    f = f_ref[:, pl.ds(i * HB, HB)]
    fb = jax.lax.bitcast_convert_type(f, jnp.int32)
    t = t_s[...]
    c = c_s[...]
    idx = i * HB + jax.lax.broadcasted_iota(jnp.int32, fb.shape, 1)
    keep = (fb > t) | ((fb == t) & (idx <= c))
    vals = jnp.where(keep, f, 0.0)
    part = jax.lax.dot_general(
        vals, w_ref[...],
        (((1,), (1,)), ((), ())),
        preferred_element_type=jnp.float32,
    )

    @pl.when(i == 0)
    def _():
        o_ref[...] = jnp.zeros_like(o_ref)

    o_ref[...] += part


@jax.jit
def kernel(x, W_enc, b_enc, W_dec):
    b2d = b_enc.reshape(1, HIDDEN_DIM)

    feats = pl.pallas_call(
        _enc_body,
        grid=(N_BLK,),
        in_specs=[
            pl.BlockSpec((BATCH, INPUT_DIM), lambda i: (0, 0)),
            pl.BlockSpec((HB, INPUT_DIM), lambda i: (i, 0)),
            pl.BlockSpec((1, HB), lambda i: (0, i)),
        ],
        out_specs=pl.BlockSpec((BATCH, HB), lambda i: (0, i)),
        out_shape=jax.ShapeDtypeStruct((BATCH, HIDDEN_DIM), jnp.float32),
    )(x, W_enc, b2d)

    recon = pl.pallas_call(
        _dec_body,
        grid=(N_BLK,),
        in_specs=[
            pl.BlockSpec((BATCH, HIDDEN_DIM), lambda i: (0, 0)),
            pl.BlockSpec((INPUT_DIM, HB), lambda i: (0, i)),
        ],
        out_specs=pl.BlockSpec((BATCH, INPUT_DIM), lambda i: (0, 0)),
        out_shape=jax.ShapeDtypeStruct((BATCH, INPUT_DIM), jnp.float32),
        scratch_shapes=[
            pltpu.VMEM((BATCH, 1), jnp.int32),
            pltpu.VMEM((BATCH, 1), jnp.int32),
        ],
    )(feats, W_dec)

    return recon
